# baseline (speedup 1.0000x reference)
import functools

import jax
import jax.numpy as jnp
from jax import lax
from jax.experimental import pallas as pl
from jax.experimental.pallas import tpu as pltpu
from jax.experimental.pallas import tpu_sc as plsc

S = 256

_NC, _NS, _NL = 2, 16, 16


def _row_block(n: int, cap: int = 2048) -> int:
    r = 8
    for c in range(8, cap + 1, 8):
        if n % c == 0:
            r = c
    return r


def _gate_body(x_ref, gw_ref, gb_ref, gate_ref):
    gate_ref[...] = jax.lax.dot_general(
        x_ref[...], gw_ref[...], (((1,), (0,)), ((), ())),
        preferred_element_type=jnp.float32) + gb_ref[...]


def _sc_stats_body(rows_per_sub, g_hbm, ids_hbm, m_hbm, s_hbm,
                   g_v, e_v, idx_v, vbuf, mxv, mx_my, stage, s_sh):
    chunk = rows_per_sub * 128
    c = lax.axis_index("c")
    sid = lax.axis_index("s")
    base_row = (c * _NS + sid) * rows_per_sub
    pltpu.sync_copy(g_hbm.at[pl.ds(base_row * 128, chunk)], g_v)
    pltpu.sync_copy(ids_hbm.at[pl.ds(base_row, rows_per_sub)], idx_v)

    def mx_body(i, acc):
        return jnp.maximum(acc, g_v[pl.ds(i * _NL, _NL)])

    loc = lax.fori_loop(0, chunk // _NL, mx_body,
                        jnp.full((_NL,), -1e30, jnp.float32))
    mx_my[...] = loc
    pltpu.sync_copy(mx_my, stage.at[pl.ds(sid * _NL, _NL)])
    plsc.subcore_barrier()
    pltpu.sync_copy(stage, mxv)

    def mx2_body(i, acc):
        return jnp.maximum(acc, mxv[pl.ds(i * _NL, _NL)])

    m_vec = lax.fori_loop(0, _NS, mx2_body,
                          jnp.full((_NL,), -1e30, jnp.float32))
    for dist in (8, 4, 2, 1):
        idx = lax.iota(jnp.int32, _NL) ^ dist
        m_vec = jnp.maximum(m_vec,
                            m_vec.at[idx].get(mode="promise_in_bounds"))

    def exp_body(i, carry):
        e_v[pl.ds(i * _NL, _NL)] = jnp.exp(g_v[pl.ds(i * _NL, _NL)] - m_vec)
        return carry

    lax.fori_loop(0, chunk // _NL, exp_body, 0)

    @pl.when(sid == 0)
    def _():
        for i in range(S // _NL):
            vbuf[pl.ds(i * _NL, _NL)] = jnp.zeros((_NL,), jnp.float32)
        pltpu.sync_copy(vbuf, s_sh)

    plsc.subcore_barrier()
    for j in range(rows_per_sub):
        pltpu.sync_copy(e_v.at[pl.ds(j * 128, 128)],
                        s_sh.at[idx_v.at[j]], add=True)
    plsc.subcore_barrier()

    @pl.when(sid == 0)
    def _():
        pltpu.sync_copy(s_sh, s_hbm.at[pl.ds(c * S, S)])
        for i in range(S // _NL):
            vbuf[pl.ds(i * _NL, _NL)] = m_vec
        pltpu.sync_copy(vbuf, m_hbm.at[pl.ds(c * S, S)])


def _sc_stats(g_flat, ids_flat):
    n = g_flat.shape[0]
    rows_per_sub = -(-n // (_NC * _NS * 128))
    rows_per_sub = -(-rows_per_sub // 8) * 8
    npad = _NC * _NS * rows_per_sub * 128
    g_pad = jnp.full((npad,), -1e30, jnp.float32).at[:n].set(g_flat)
    ids_pad = jnp.zeros((npad,), jnp.int32).at[:n].set(ids_flat)
    ids2d = ids_pad.reshape(npad // 128, 128)
    mesh = plsc.VectorSubcoreMesh(core_axis_name="c", subcore_axis_name="s",
                                  num_cores=_NC, num_subcores=_NS)
    m_out, s_out = pl.kernel(
        functools.partial(_sc_stats_body, rows_per_sub),
        out_type=[
            jax.ShapeDtypeStruct((_NC * S,), jnp.float32),
            jax.ShapeDtypeStruct((_NC * S,), jnp.float32),
        ],
        mesh=mesh,
        scratch_types=[
            pltpu.VMEM((rows_per_sub * 128,), jnp.float32),
            pltpu.VMEM((rows_per_sub * 128,), jnp.float32),
            pltpu.VMEM((rows_per_sub, 128), jnp.int32),
            pltpu.VMEM((S,), jnp.float32),
            pltpu.VMEM((_NS * _NL,), jnp.float32),
            pltpu.VMEM((_NL,), jnp.float32),
            pltpu.VMEM_SHARED((_NS * _NL,), jnp.float32),
            pltpu.VMEM_SHARED((S,), jnp.float32),
        ],
    )(g_pad, ids2d)
    return m_out.reshape(_NC, S), s_out.reshape(_NC, S)


def _main_body(x_ref, w1_ref, b1_ref, a_ref, ids_ref, gate_ref, m_ref, s_ref,
               acc_ref):
    i = pl.program_id(0)
    e = jnp.exp(gate_ref[...] - m_ref[...])
    oh = (ids_ref[...] == jax.lax.broadcasted_iota(jnp.int32, (1, S), 1))
    ohf = oh.astype(jnp.float32)
    sg = jnp.sum(ohf * s_ref[...], axis=1, keepdims=True)
    w = e / (sg + 1e-16)
    u = jax.lax.dot_general(x_ref[...], w1_ref[...], (((1,), (0,)), ((), ())),
                            preferred_element_type=jnp.float32) + b1_ref[...]
    u = jnp.where(u >= 0, u, a_ref[...] * u)
    contrib = jax.lax.dot_general(ohf * w, u, (((0,), (0,)), ((), ())),
                                  preferred_element_type=jnp.float32)

    @pl.when(i == 0)
    def _():
        acc_ref[...] = contrib

    @pl.when(i > 0)
    def _():
        acc_ref[...] += contrib


def _epilogue_body(aa_ref, ab_ref, am_ref, sc_ref, gsc_ref, beta_ref, w2_ref,
                   b2_ref, out_ref):
    acc = aa_ref[...] + ab_ref[...] + am_ref[...]
    sw = sc_ref[...] / (sc_ref[...] + 1e-16)
    z = acc * gsc_ref[...] + sw * beta_ref[...]
    out_ref[...] = jax.lax.dot_general(
        z, w2_ref[...], (((1,), (0,)), ((), ())),
        preferred_element_type=jnp.float32) + sw * b2_ref[...]


def _gate(x, gw_col, gb11):
    n, d = x.shape
    r = _row_block(n)
    nb = n // r
    return pl.pallas_call(
        _gate_body,
        grid=(nb,),
        in_specs=[
            pl.BlockSpec((r, d), lambda i: (i, 0)),
            pl.BlockSpec((d, 1), lambda i: (0, 0)),
            pl.BlockSpec((1, 1), lambda i: (0, 0)),
        ],
        out_specs=pl.BlockSpec((r, 1), lambda i: (i, 0)),
        out_shape=jax.ShapeDtypeStruct((n, 1), jnp.float32),
        compiler_params=pltpu.CompilerParams(
            dimension_semantics=("arbitrary",)),
    )(x, gw_col, gb11)


def _main(x, ids_col, gate_col, w1, b1_row, a11, m11, s_row):
    n, d = x.shape
    r = _row_block(n)
    nb = n // r
    return pl.pallas_call(
        _main_body,
        grid=(nb,),
        in_specs=[
            pl.BlockSpec((r, d), lambda i: (i, 0)),
            pl.BlockSpec((d, d), lambda i: (0, 0)),
            pl.BlockSpec((1, d), lambda i: (0, 0)),
            pl.BlockSpec((1, 1), lambda i: (0, 0)),
            pl.BlockSpec((r, 1), lambda i: (i, 0)),
            pl.BlockSpec((r, 1), lambda i: (i, 0)),
            pl.BlockSpec((1, 1), lambda i: (0, 0)),
            pl.BlockSpec((1, S), lambda i: (0, 0)),
        ],
        out_specs=pl.BlockSpec((S, d), lambda i: (0, 0)),
        out_shape=jax.ShapeDtypeStruct((S, d), jnp.float32),
        compiler_params=pltpu.CompilerParams(
            dimension_semantics=("arbitrary",)),
    )(x, w1, b1_row, a11, ids_col, gate_col, m11, s_row)


def kernel(nodes_atoms, nodes_bonds, nodes_monosacchs, batch_ids_atoms,
           batch_ids_bonds, batch_ids_monosacchs, gate_W, gate_b, W1, b1,
           prelu_a, bn_gamma, bn_beta, W2, b2):
    d = nodes_atoms.shape[1]
    parts = [
        (nodes_atoms, batch_ids_atoms),
        (nodes_bonds, batch_ids_bonds),
        (nodes_monosacchs, batch_ids_monosacchs),
    ]
    gw_col = gate_W.reshape(d, 1).astype(jnp.float32)
    gb11 = gate_b.reshape(1, 1).astype(jnp.float32)
    a11 = jnp.asarray(prelu_a, jnp.float32).reshape(1, 1)
    b1_row = b1.reshape(1, d)
    b2_row = b2.reshape(1, d)
    beta_row = bn_beta.reshape(1, d)
    gscale_row = (bn_gamma * (1.0 / jnp.sqrt(1.0 + 1e-5))).reshape(1, d)

    ids_cols, gates = [], []
    for x, ids in parts:
        ids_cols.append(ids.astype(jnp.int32).reshape(-1, 1))
        gates.append(_gate(x, gw_col, gb11))

    g_flat = jnp.concatenate([g.reshape(-1) for g in gates])
    ids_flat = jnp.concatenate([i.reshape(-1) for i in ids_cols])
    m_out, s_out = _sc_stats(g_flat, ids_flat)

    m_glob = jnp.max(m_out).reshape(1, 1)
    s_tot = jnp.sum(s_out * jnp.exp(m_out - m_glob[0, 0]),
                    axis=0).reshape(1, S)

    accs = [
        _main(x, ids_cols[k], gates[k], W1, b1_row, a11, m_glob, s_tot)
        for k, (x, _) in enumerate(parts)
    ]

    s_col = s_tot.reshape(S, 1)
    out = pl.pallas_call(
        _epilogue_body,
        out_shape=jax.ShapeDtypeStruct((S, d), jnp.float32),
    )(accs[0], accs[1], accs[2], s_col, gscale_row, beta_row, W2, b2_row)
    return out


# speedup vs baseline: 1.0035x; 1.0035x over previous
import functools

import jax
import jax.numpy as jnp
from jax import lax
from jax.experimental import pallas as pl
from jax.experimental.pallas import tpu as pltpu
from jax.experimental.pallas import tpu_sc as plsc

S = 256

_NC, _NS, _NL = 2, 16, 16


def _row_block(n: int, cap: int = 2048) -> int:
    r = 8
    for c in range(8, cap + 1, 8):
        if n % c == 0:
            r = c
    return r


def _gate_body(x_ref, gw_ref, gb_ref, gate_ref):
    gate_ref[...] = jax.lax.dot_general(
        x_ref[...], gw_ref[...], (((1,), (0,)), ((), ())),
        preferred_element_type=jnp.float32) + gb_ref[...]


def _sc_stats_body(rows_per_sub, g_hbm, ids_hbm, m_hbm, s_hbm,
                   g_v, e_v, idx_v, vbuf, mxv, mx_my, stage, s_sh):
    chunk = rows_per_sub * 128
    c = lax.axis_index("c")
    sid = lax.axis_index("s")
    base_row = (c * _NS + sid) * rows_per_sub
    pltpu.sync_copy(g_hbm.at[pl.ds(base_row * 128, chunk)], g_v)
    pltpu.sync_copy(ids_hbm.at[pl.ds(base_row, rows_per_sub)], idx_v)

    def mx_body(i, acc):
        return jnp.maximum(acc, g_v[pl.ds(i * _NL, _NL)])

    loc = lax.fori_loop(0, chunk // _NL, mx_body,
                        jnp.full((_NL,), -1e30, jnp.float32))
    mx_my[...] = loc
    pltpu.sync_copy(mx_my, stage.at[pl.ds(sid * _NL, _NL)])
    plsc.subcore_barrier()
    pltpu.sync_copy(stage, mxv)

    def mx2_body(i, acc):
        return jnp.maximum(acc, mxv[pl.ds(i * _NL, _NL)])

    m_vec = lax.fori_loop(0, _NS, mx2_body,
                          jnp.full((_NL,), -1e30, jnp.float32))
    for dist in (8, 4, 2, 1):
        idx = lax.iota(jnp.int32, _NL) ^ dist
        m_vec = jnp.maximum(m_vec,
                            m_vec.at[idx].get(mode="promise_in_bounds"))

    def exp_body(i, carry):
        e_v[pl.ds(i * _NL, _NL)] = jnp.exp(g_v[pl.ds(i * _NL, _NL)] - m_vec)
        return carry

    lax.fori_loop(0, chunk // _NL, exp_body, 0)

    @pl.when(sid == 0)
    def _():
        for i in range(S // _NL):
            vbuf[pl.ds(i * _NL, _NL)] = jnp.zeros((_NL,), jnp.float32)
        pltpu.sync_copy(vbuf, s_sh)

    plsc.subcore_barrier()
    for j in range(rows_per_sub):
        pltpu.sync_copy(e_v.at[pl.ds(j * 128, 128)],
                        s_sh.at[idx_v.at[j]], add=True)
    plsc.subcore_barrier()

    @pl.when(sid == 0)
    def _():
        pltpu.sync_copy(s_sh, s_hbm.at[pl.ds(c * S, S)])
        for i in range(S // _NL):
            vbuf[pl.ds(i * _NL, _NL)] = m_vec
        pltpu.sync_copy(vbuf, m_hbm.at[pl.ds(c * S, S)])


def _sc_stats(g_flat, ids_flat):
    n = g_flat.shape[0]
    rows_per_sub = -(-n // (_NC * _NS * 128))
    rows_per_sub = -(-rows_per_sub // 8) * 8
    npad = _NC * _NS * rows_per_sub * 128
    g_pad = jnp.full((npad,), -1e30, jnp.float32).at[:n].set(g_flat)
    ids_pad = jnp.zeros((npad,), jnp.int32).at[:n].set(ids_flat)
    ids2d = ids_pad.reshape(npad // 128, 128)
    mesh = plsc.VectorSubcoreMesh(core_axis_name="c", subcore_axis_name="s",
                                  num_cores=_NC, num_subcores=_NS)
    m_out, s_out = pl.kernel(
        functools.partial(_sc_stats_body, rows_per_sub),
        out_type=[
            jax.ShapeDtypeStruct((_NC * S,), jnp.float32),
            jax.ShapeDtypeStruct((_NC * S,), jnp.float32),
        ],
        mesh=mesh,
        scratch_types=[
            pltpu.VMEM((rows_per_sub * 128,), jnp.float32),
            pltpu.VMEM((rows_per_sub * 128,), jnp.float32),
            pltpu.VMEM((rows_per_sub, 128), jnp.int32),
            pltpu.VMEM((S,), jnp.float32),
            pltpu.VMEM((_NS * _NL,), jnp.float32),
            pltpu.VMEM((_NL,), jnp.float32),
            pltpu.VMEM_SHARED((_NS * _NL,), jnp.float32),
            pltpu.VMEM_SHARED((S,), jnp.float32),
        ],
    )(g_pad, ids2d)
    return m_out.reshape(_NC, S), s_out.reshape(_NC, S)


def _main_body(x_ref, w1_ref, b1_ref, a_ref, ids_ref, gate_ref, m_ref, s_ref,
               acc_ref):
    i = pl.program_id(0)
    e = jnp.exp(gate_ref[...] - m_ref[...])
    oh = (ids_ref[...] == jax.lax.broadcasted_iota(jnp.int32, (1, S), 1))
    ohf = oh.astype(jnp.float32)
    sg = jnp.sum(ohf * s_ref[...], axis=1, keepdims=True)
    w = e / (sg + 1e-16)
    u = jax.lax.dot_general(x_ref[...].astype(jnp.bfloat16), w1_ref[...],
                            (((1,), (0,)), ((), ())),
                            preferred_element_type=jnp.float32) + b1_ref[...]
    u = jnp.where(u >= 0, u, a_ref[...] * u)
    contrib = jax.lax.dot_general(
        (ohf * w).astype(jnp.bfloat16), u.astype(jnp.bfloat16),
        (((0,), (0,)), ((), ())),
        preferred_element_type=jnp.float32)

    @pl.when(i == 0)
    def _():
        acc_ref[...] = contrib

    @pl.when(i > 0)
    def _():
        acc_ref[...] += contrib


def _epilogue_body(aa_ref, ab_ref, am_ref, sc_ref, gsc_ref, beta_ref, w2_ref,
                   b2_ref, out_ref):
    acc = aa_ref[...] + ab_ref[...] + am_ref[...]
    sw = sc_ref[...] / (sc_ref[...] + 1e-16)
    z = acc * gsc_ref[...] + sw * beta_ref[...]
    out_ref[...] = jax.lax.dot_general(
        z, w2_ref[...], (((1,), (0,)), ((), ())),
        preferred_element_type=jnp.float32) + sw * b2_ref[...]


def _gate(x, gw_col, gb11):
    n, d = x.shape
    r = _row_block(n)
    nb = n // r
    return pl.pallas_call(
        _gate_body,
        grid=(nb,),
        in_specs=[
            pl.BlockSpec((r, d), lambda i: (i, 0)),
            pl.BlockSpec((d, 1), lambda i: (0, 0)),
            pl.BlockSpec((1, 1), lambda i: (0, 0)),
        ],
        out_specs=pl.BlockSpec((r, 1), lambda i: (i, 0)),
        out_shape=jax.ShapeDtypeStruct((n, 1), jnp.float32),
        compiler_params=pltpu.CompilerParams(
            dimension_semantics=("arbitrary",)),
    )(x, gw_col, gb11)


def _main(x, ids_col, gate_col, w1, b1_row, a11, m11, s_row):
    n, d = x.shape
    r = _row_block(n)
    nb = n // r
    return pl.pallas_call(
        _main_body,
        grid=(nb,),
        in_specs=[
            pl.BlockSpec((r, d), lambda i: (i, 0)),
            pl.BlockSpec((d, d), lambda i: (0, 0)),
            pl.BlockSpec((1, d), lambda i: (0, 0)),
            pl.BlockSpec((1, 1), lambda i: (0, 0)),
            pl.BlockSpec((r, 1), lambda i: (i, 0)),
            pl.BlockSpec((r, 1), lambda i: (i, 0)),
            pl.BlockSpec((1, 1), lambda i: (0, 0)),
            pl.BlockSpec((1, S), lambda i: (0, 0)),
        ],
        out_specs=pl.BlockSpec((S, d), lambda i: (0, 0)),
        out_shape=jax.ShapeDtypeStruct((S, d), jnp.float32),
        compiler_params=pltpu.CompilerParams(
            dimension_semantics=("arbitrary",)),
    )(x, w1, b1_row, a11, ids_col, gate_col, m11, s_row)


def kernel(nodes_atoms, nodes_bonds, nodes_monosacchs, batch_ids_atoms,
           batch_ids_bonds, batch_ids_monosacchs, gate_W, gate_b, W1, b1,
           prelu_a, bn_gamma, bn_beta, W2, b2):
    d = nodes_atoms.shape[1]
    parts = [
        (nodes_atoms, batch_ids_atoms),
        (nodes_bonds, batch_ids_bonds),
        (nodes_monosacchs, batch_ids_monosacchs),
    ]
    gw_col = gate_W.reshape(d, 1).astype(jnp.float32)
    gb11 = gate_b.reshape(1, 1).astype(jnp.float32)
    a11 = jnp.asarray(prelu_a, jnp.float32).reshape(1, 1)
    b1_row = b1.reshape(1, d)
    b2_row = b2.reshape(1, d)
    beta_row = bn_beta.reshape(1, d)
    gscale_row = (bn_gamma * (1.0 / jnp.sqrt(1.0 + 1e-5))).reshape(1, d)

    ids_cols, gates = [], []
    for x, ids in parts:
        ids_cols.append(ids.astype(jnp.int32).reshape(-1, 1))
        gates.append(_gate(x, gw_col, gb11))

    g_flat = jnp.concatenate([g.reshape(-1) for g in gates])
    ids_flat = jnp.concatenate([i.reshape(-1) for i in ids_cols])
    m_out, s_out = _sc_stats(g_flat, ids_flat)

    m_glob = jnp.max(m_out).reshape(1, 1)
    s_tot = jnp.sum(s_out * jnp.exp(m_out - m_glob[0, 0]),
                    axis=0).reshape(1, S)

    w1_bf = W1.astype(jnp.bfloat16)
    accs = [
        _main(x, ids_cols[k], gates[k], w1_bf, b1_row, a11, m_glob, s_tot)
        for k, (x, _) in enumerate(parts)
    ]

    s_col = s_tot.reshape(S, 1)
    out = pl.pallas_call(
        _epilogue_body,
        out_shape=jax.ShapeDtypeStruct((S, d), jnp.float32),
    )(accs[0], accs[1], accs[2], s_col, gscale_row, beta_row, W2, b2_row)
    return out


# speedup vs baseline: 1.0702x; 1.0665x over previous
import functools

import jax
import jax.numpy as jnp
from jax import lax
from jax.experimental import pallas as pl
from jax.experimental.pallas import tpu as pltpu
from jax.experimental.pallas import tpu_sc as plsc

S = 256
R = 1000

_NC, _NS, _NL = 2, 16, 16

_ARB = pltpu.CompilerParams(dimension_semantics=("arbitrary",))


def _gate(x_ref, gw_ref, gb_ref):
    return jax.lax.dot_general(x_ref[...], gw_ref[...], (((1,), (0,)), ((), ())),
                               preferred_element_type=jnp.float32) + gb_ref[...]


def _wcontrib(x_ref, w1_ref, b1_ref, a_ref, ids_ref, gate_ref, m_ref):
    e = jnp.exp(gate_ref[...] - m_ref[...])
    ohw = (ids_ref[...] == jax.lax.broadcasted_iota(jnp.int32, (1, S), 1)
           ).astype(jnp.float32) * e
    u = jax.lax.dot_general(x_ref[...].astype(jnp.bfloat16), w1_ref[...],
                            (((1,), (0,)), ((), ())),
                            preferred_element_type=jnp.float32) + b1_ref[...]
    u = jnp.where(u >= 0, u, a_ref[...] * u)
    return jax.lax.dot_general(ohw.astype(jnp.bfloat16), u.astype(jnp.bfloat16),
                               (((0,), (0,)), ((), ())),
                               preferred_element_type=jnp.float32)


def _k1_body(nbm, xa_ref, xm_ref, gw_ref, gb_ref, ga_ref, gm_ref):
    i = pl.program_id(0)
    ga_ref[...] = _gate(xa_ref, gw_ref, gb_ref)

    @pl.when(i < nbm)
    def _():
        gm_ref[...] = _gate(xm_ref, gw_ref, gb_ref)


def _k2_body(xa_ref, w1_ref, b1_ref, a_ref, ids_ref, gate_ref, m_ref,
             xb_ref, gw_ref, gb_ref, acc_ref, gb_out_ref):
    i = pl.program_id(0)
    gb_out_ref[...] = _gate(xb_ref, gw_ref, gb_ref)
    contrib = _wcontrib(xa_ref, w1_ref, b1_ref, a_ref, ids_ref, gate_ref, m_ref)

    @pl.when(i == 0)
    def _():
        acc_ref[...] = contrib

    @pl.when(i > 0)
    def _():
        acc_ref[...] += contrib


def _k3_body(nbm, xb_ref, w1_ref, b1_ref, a_ref, idsb_ref, gateb_ref, mb_ref,
             xm_ref, idsm_ref, gatem_ref, mm_ref, accb_ref, accm_ref):
    i = pl.program_id(0)
    contrib_b = _wcontrib(xb_ref, w1_ref, b1_ref, a_ref, idsb_ref, gateb_ref,
                          mb_ref)

    @pl.when(i == 0)
    def _():
        accb_ref[...] = contrib_b

    @pl.when(i > 0)
    def _():
        accb_ref[...] += contrib_b

    @pl.when(i < nbm)
    def _():
        contrib_m = _wcontrib(xm_ref, w1_ref, b1_ref, a_ref, idsm_ref,
                              gatem_ref, mm_ref)

        @pl.when(i == 0)
        def _():
            accm_ref[...] = contrib_m

        @pl.when(i > 0)
        def _():
            accm_ref[...] += contrib_m


def _epilogue_body(aa_ref, ab_ref, am_ref, kam_ref, kb_ref, den_ref,
                   gsc_ref, beta_ref, w2_ref, b2_ref, out_ref):
    acc = kam_ref[...] * (aa_ref[...] + am_ref[...]) + kb_ref[...] * ab_ref[...]
    den = den_ref[...] + 1e-16
    sumw = den_ref[...] / den
    z = (acc / den) * gsc_ref[...] + sumw * beta_ref[...]
    out_ref[...] = jax.lax.dot_general(
        z, w2_ref[...], (((1,), (0,)), ((), ())),
        preferred_element_type=jnp.float32) + sumw * b2_ref[...]


def _sc_stats_body(rows_per_sub, g_hbm, ids_hbm, m_hbm, s_hbm,
                   g_v, e_v, idx_v, vbuf, mxv, mx_my, stage, s_sh):
    chunk = rows_per_sub * 128
    c = lax.axis_index("c")
    sid = lax.axis_index("s")
    base_row = (c * _NS + sid) * rows_per_sub
    pltpu.sync_copy(g_hbm.at[pl.ds(base_row * 128, chunk)], g_v)
    pltpu.sync_copy(ids_hbm.at[pl.ds(base_row, rows_per_sub)], idx_v)

    def mx_body(i, acc):
        return jnp.maximum(acc, g_v[pl.ds(i * _NL, _NL)])

    loc = lax.fori_loop(0, chunk // _NL, mx_body,
                        jnp.full((_NL,), -1e30, jnp.float32))
    mx_my[...] = loc
    pltpu.sync_copy(mx_my, stage.at[pl.ds(sid * _NL, _NL)])
    plsc.subcore_barrier()
    pltpu.sync_copy(stage, mxv)

    def mx2_body(i, acc):
        return jnp.maximum(acc, mxv[pl.ds(i * _NL, _NL)])

    m_vec = lax.fori_loop(0, _NS, mx2_body,
                          jnp.full((_NL,), -1e30, jnp.float32))
    for dist in (8, 4, 2, 1):
        idx = lax.iota(jnp.int32, _NL) ^ dist
        m_vec = jnp.maximum(m_vec,
                            m_vec.at[idx].get(mode="promise_in_bounds"))

    def exp_body(i, carry):
        e_v[pl.ds(i * _NL, _NL)] = jnp.exp(g_v[pl.ds(i * _NL, _NL)] - m_vec)
        return carry

    lax.fori_loop(0, chunk // _NL, exp_body, 0)

    @pl.when(sid == 0)
    def _():
        for i in range(S // _NL):
            vbuf[pl.ds(i * _NL, _NL)] = jnp.zeros((_NL,), jnp.float32)
        pltpu.sync_copy(vbuf, s_sh)

    plsc.subcore_barrier()
    for j in range(rows_per_sub):
        pltpu.sync_copy(e_v.at[pl.ds(j * 128, 128)],
                        s_sh.at[idx_v.at[j]], add=True)
    plsc.subcore_barrier()

    @pl.when(sid == 0)
    def _():
        pltpu.sync_copy(s_sh, s_hbm.at[pl.ds(c * S, S)])
        for i in range(S // _NL):
            vbuf[pl.ds(i * _NL, _NL)] = m_vec
        pltpu.sync_copy(vbuf, m_hbm.at[pl.ds(c * S, S)])


def _sc_stats(g_flat, ids_flat):
    n = g_flat.shape[0]
    rows_per_sub = -(-n // (_NC * _NS * 128))
    rows_per_sub = -(-rows_per_sub // 8) * 8
    npad = _NC * _NS * rows_per_sub * 128
    g_pad = jnp.full((npad,), -1e30, jnp.float32).at[:n].set(g_flat)
    ids_pad = jnp.zeros((npad,), jnp.int32).at[:n].set(ids_flat)
    ids2d = ids_pad.reshape(npad // 128, 128)
    mesh = plsc.VectorSubcoreMesh(core_axis_name="c", subcore_axis_name="s",
                                  num_cores=_NC, num_subcores=_NS)
    m_out, s_out = pl.kernel(
        functools.partial(_sc_stats_body, rows_per_sub),
        out_type=[
            jax.ShapeDtypeStruct((_NC * S,), jnp.float32),
            jax.ShapeDtypeStruct((_NC * S,), jnp.float32),
        ],
        mesh=mesh,
        scratch_types=[
            pltpu.VMEM((rows_per_sub * 128,), jnp.float32),
            pltpu.VMEM((rows_per_sub * 128,), jnp.float32),
            pltpu.VMEM((rows_per_sub, 128), jnp.int32),
            pltpu.VMEM((S,), jnp.float32),
            pltpu.VMEM((_NS * _NL,), jnp.float32),
            pltpu.VMEM((_NL,), jnp.float32),
            pltpu.VMEM_SHARED((_NS * _NL,), jnp.float32),
            pltpu.VMEM_SHARED((S,), jnp.float32),
        ],
    )(g_pad, ids2d)
    return m_out.reshape(_NC, S), s_out.reshape(_NC, S)


def _merge_cores(m_out, s_out):
    m_g = jnp.max(m_out)
    s_g = jnp.sum(s_out * jnp.exp(m_out - m_g), axis=0)
    return m_g, s_g


def kernel(nodes_atoms, nodes_bonds, nodes_monosacchs, batch_ids_atoms,
           batch_ids_bonds, batch_ids_monosacchs, gate_W, gate_b, W1, b1,
           prelu_a, bn_gamma, bn_beta, W2, b2):
    d = nodes_atoms.shape[1]
    na, nb, nm = (nodes_atoms.shape[0], nodes_bonds.shape[0],
                  nodes_monosacchs.shape[0])
    nba, nbb, nbm = na // R, nb // R, nm // R
    gw_col = gate_W.reshape(d, 1).astype(jnp.float32)
    gb11 = gate_b.reshape(1, 1).astype(jnp.float32)
    a11 = jnp.asarray(prelu_a, jnp.float32).reshape(1, 1)
    b1_row = b1.reshape(1, d)
    b2_row = b2.reshape(1, d)
    beta_row = bn_beta.reshape(1, d)
    gscale_row = (bn_gamma * (1.0 / jnp.sqrt(1.0 + 1e-5))).reshape(1, d)
    w1_bf = W1.astype(jnp.bfloat16)
    ids_a = batch_ids_atoms.astype(jnp.int32)
    ids_b = batch_ids_bonds.astype(jnp.int32)
    ids_m = batch_ids_monosacchs.astype(jnp.int32)

    blk = lambda shp, im: pl.BlockSpec(shp, im)
    row0 = lambda i: (i, 0)
    const0 = lambda i: (0, 0)
    rowm = lambda i: (jnp.minimum(i, nbm - 1), 0)

    gate_a, gate_m = pl.pallas_call(
        functools.partial(_k1_body, nbm),
        grid=(nba,),
        in_specs=[blk((R, d), row0), blk((R, d), rowm),
                  blk((d, 1), const0), blk((1, 1), const0)],
        out_specs=[blk((R, 1), row0), blk((R, 1), rowm)],
        out_shape=[jax.ShapeDtypeStruct((na, 1), jnp.float32),
                   jax.ShapeDtypeStruct((nm, 1), jnp.float32)],
        compiler_params=_ARB,
    )(nodes_atoms, nodes_monosacchs, gw_col, gb11)

    m_am_out, s_am_out = _sc_stats(
        jnp.concatenate([gate_a.reshape(-1), gate_m.reshape(-1)]),
        jnp.concatenate([ids_a, ids_m]))
    m_am, s_am = _merge_cores(m_am_out, s_am_out)
    m_am11 = m_am.reshape(1, 1)

    acc_a, gate_b_col = pl.pallas_call(
        _k2_body,
        grid=(nba,),
        in_specs=[blk((R, d), row0), blk((d, d), const0), blk((1, d), const0),
                  blk((1, 1), const0), blk((R, 1), row0), blk((R, 1), row0),
                  blk((1, 1), const0),
                  blk((R, d), row0), blk((d, 1), const0), blk((1, 1), const0)],
        out_specs=[blk((S, d), const0), blk((R, 1), row0)],
        out_shape=[jax.ShapeDtypeStruct((S, d), jnp.float32),
                   jax.ShapeDtypeStruct((nb, 1), jnp.float32)],
        compiler_params=_ARB,
    )(nodes_atoms, w1_bf, b1_row, a11, ids_a.reshape(na, 1), gate_a, m_am11,
      nodes_bonds, gw_col, gb11)

    m_b_out, s_b_out = _sc_stats(gate_b_col.reshape(-1), ids_b)
    m_b, s_b = _merge_cores(m_b_out, s_b_out)
    m_b11 = m_b.reshape(1, 1)

    acc_b, acc_m = pl.pallas_call(
        functools.partial(_k3_body, nbm),
        grid=(nbb,),
        in_specs=[blk((R, d), row0), blk((d, d), const0), blk((1, d), const0),
                  blk((1, 1), const0), blk((R, 1), row0), blk((R, 1), row0),
                  blk((1, 1), const0),
                  blk((R, d), rowm), blk((R, 1), rowm), blk((R, 1), rowm),
                  blk((1, 1), const0)],
        out_specs=[blk((S, d), const0), blk((S, d), const0)],
        out_shape=[jax.ShapeDtypeStruct((S, d), jnp.float32),
                   jax.ShapeDtypeStruct((S, d), jnp.float32)],
        compiler_params=_ARB,
    )(nodes_bonds, w1_bf, b1_row, a11, ids_b.reshape(nb, 1), gate_b_col,
      m_b11, nodes_monosacchs, ids_m.reshape(nm, 1), gate_m, m_am11)

    m_glob = jnp.maximum(m_am, m_b)
    k_am = jnp.exp(m_am - m_glob).reshape(1, 1)
    k_b = jnp.exp(m_b - m_glob).reshape(1, 1)
    den_col = (k_am[0, 0] * s_am + k_b[0, 0] * s_b).reshape(S, 1)

    out = pl.pallas_call(
        _epilogue_body,
        out_shape=jax.ShapeDtypeStruct((S, d), jnp.float32),
    )(acc_a, acc_b, acc_m, k_am, k_b, den_col, gscale_row, beta_row, W2,
      b2_row)
    return out


# speedup vs baseline: 1.0745x; 1.0040x over previous
import functools

import jax
import jax.numpy as jnp
from jax import lax
from jax.experimental import pallas as pl
from jax.experimental.pallas import tpu as pltpu
from jax.experimental.pallas import tpu_sc as plsc

S = 256
R = 1000

_NC, _NS, _NL = 2, 16, 16

_ARB = pltpu.CompilerParams(dimension_semantics=("arbitrary",))


def _gate(x_ref, gw_ref, gb_ref):
    return jax.lax.dot_general(x_ref[...], gw_ref[...], (((1,), (0,)), ((), ())),
                               preferred_element_type=jnp.float32) + gb_ref[...]


def _wcontrib(x_ref, w1_ref, b1_ref, a_ref, ids_ref, gate_ref, m_ref):
    e = jnp.exp(gate_ref[...] - m_ref[...])
    eq = ids_ref[...] == jax.lax.broadcasted_iota(jnp.int32, (1, S), 1)
    ohw = (eq.astype(jnp.float32) * e).astype(jnp.bfloat16)
    u = jax.lax.dot_general(x_ref[...].astype(jnp.bfloat16), w1_ref[...],
                            (((1,), (0,)), ((), ())),
                            preferred_element_type=jnp.float32
                            ).astype(jnp.bfloat16) + b1_ref[...]
    u = jnp.where(u >= 0, u, a_ref[...] * u)
    return jax.lax.dot_general(ohw, u, (((0,), (0,)), ((), ())),
                               preferred_element_type=jnp.float32)


def _k1_body(nbm, xa_ref, xm_ref, gw_ref, gb_ref, ga_ref, gm_ref):
    i = pl.program_id(0)
    ga_ref[...] = _gate(xa_ref, gw_ref, gb_ref)

    @pl.when(i < nbm)
    def _():
        gm_ref[...] = _gate(xm_ref, gw_ref, gb_ref)


def _k2_body(xa_ref, w1_ref, b1_ref, a_ref, ids_ref, gate_ref, m_ref,
             xb_ref, gw_ref, gb_ref, acc_ref, gb_out_ref):
    i = pl.program_id(0)
    gb_out_ref[...] = _gate(xb_ref, gw_ref, gb_ref)
    contrib = _wcontrib(xa_ref, w1_ref, b1_ref, a_ref, ids_ref, gate_ref, m_ref)

    @pl.when(i == 0)
    def _():
        acc_ref[...] = contrib

    @pl.when(i > 0)
    def _():
        acc_ref[...] += contrib


def _k3_body(nbm, xb_ref, w1_ref, b1_ref, a_ref, idsb_ref, gateb_ref, mb_ref,
             xm_ref, idsm_ref, gatem_ref, mm_ref, accb_ref, accm_ref):
    i = pl.program_id(0)
    contrib_b = _wcontrib(xb_ref, w1_ref, b1_ref, a_ref, idsb_ref, gateb_ref,
                          mb_ref)

    @pl.when(i == 0)
    def _():
        accb_ref[...] = contrib_b

    @pl.when(i > 0)
    def _():
        accb_ref[...] += contrib_b

    @pl.when(i < nbm)
    def _():
        contrib_m = _wcontrib(xm_ref, w1_ref, b1_ref, a_ref, idsm_ref,
                              gatem_ref, mm_ref)

        @pl.when(i == 0)
        def _():
            accm_ref[...] = contrib_m

        @pl.when(i > 0)
        def _():
            accm_ref[...] += contrib_m


def _epilogue_body(aa_ref, ab_ref, am_ref, kam_ref, kb_ref, den_ref,
                   gsc_ref, beta_ref, w2_ref, b2_ref, out_ref):
    acc = kam_ref[...] * (aa_ref[...] + am_ref[...]) + kb_ref[...] * ab_ref[...]
    den = den_ref[...] + 1e-16
    sumw = den_ref[...] / den
    z = (acc / den) * gsc_ref[...] + sumw * beta_ref[...]
    out_ref[...] = jax.lax.dot_general(
        z, w2_ref[...], (((1,), (0,)), ((), ())),
        preferred_element_type=jnp.float32) + sumw * b2_ref[...]


def _sc_stats_body(rows_per_sub, g_hbm, ids_hbm, m_hbm, s_hbm,
                   g_v, e_v, idx_v, vbuf, mxv, mx_my, stage, s_sh):
    chunk = rows_per_sub * 128
    c = lax.axis_index("c")
    sid = lax.axis_index("s")
    base_row = (c * _NS + sid) * rows_per_sub
    pltpu.sync_copy(g_hbm.at[pl.ds(base_row * 128, chunk)], g_v)
    pltpu.sync_copy(ids_hbm.at[pl.ds(base_row, rows_per_sub)], idx_v)

    def mx_body(i, acc):
        return jnp.maximum(acc, g_v[pl.ds(i * _NL, _NL)])

    loc = lax.fori_loop(0, chunk // _NL, mx_body,
                        jnp.full((_NL,), -1e30, jnp.float32))
    mx_my[...] = loc
    pltpu.sync_copy(mx_my, stage.at[pl.ds(sid * _NL, _NL)])
    plsc.subcore_barrier()
    pltpu.sync_copy(stage, mxv)

    def mx2_body(i, acc):
        return jnp.maximum(acc, mxv[pl.ds(i * _NL, _NL)])

    m_vec = lax.fori_loop(0, _NS, mx2_body,
                          jnp.full((_NL,), -1e30, jnp.float32))
    for dist in (8, 4, 2, 1):
        idx = lax.iota(jnp.int32, _NL) ^ dist
        m_vec = jnp.maximum(m_vec,
                            m_vec.at[idx].get(mode="promise_in_bounds"))

    def exp_body(i, carry):
        e_v[pl.ds(i * _NL, _NL)] = jnp.exp(g_v[pl.ds(i * _NL, _NL)] - m_vec)
        return carry

    lax.fori_loop(0, chunk // _NL, exp_body, 0)

    @pl.when(sid == 0)
    def _():
        for i in range(S // _NL):
            vbuf[pl.ds(i * _NL, _NL)] = jnp.zeros((_NL,), jnp.float32)
        pltpu.sync_copy(vbuf, s_sh)

    plsc.subcore_barrier()
    for j in range(rows_per_sub):
        pltpu.sync_copy(e_v.at[pl.ds(j * 128, 128)],
                        s_sh.at[idx_v.at[j]], add=True)
    plsc.subcore_barrier()

    @pl.when(sid == 0)
    def _():
        pltpu.sync_copy(s_sh, s_hbm.at[pl.ds(c * S, S)])
        for i in range(S // _NL):
            vbuf[pl.ds(i * _NL, _NL)] = m_vec
        pltpu.sync_copy(vbuf, m_hbm.at[pl.ds(c * S, S)])


def _sc_stats(g_flat, ids_flat):
    n = g_flat.shape[0]
    rows_per_sub = -(-n // (_NC * _NS * 128))
    rows_per_sub = -(-rows_per_sub // 8) * 8
    npad = _NC * _NS * rows_per_sub * 128
    g_pad = jnp.full((npad,), -1e30, jnp.float32).at[:n].set(g_flat)
    ids_pad = jnp.zeros((npad,), jnp.int32).at[:n].set(ids_flat)
    ids2d = ids_pad.reshape(npad // 128, 128)
    mesh = plsc.VectorSubcoreMesh(core_axis_name="c", subcore_axis_name="s",
                                  num_cores=_NC, num_subcores=_NS)
    m_out, s_out = pl.kernel(
        functools.partial(_sc_stats_body, rows_per_sub),
        out_type=[
            jax.ShapeDtypeStruct((_NC * S,), jnp.float32),
            jax.ShapeDtypeStruct((_NC * S,), jnp.float32),
        ],
        mesh=mesh,
        scratch_types=[
            pltpu.VMEM((rows_per_sub * 128,), jnp.float32),
            pltpu.VMEM((rows_per_sub * 128,), jnp.float32),
            pltpu.VMEM((rows_per_sub, 128), jnp.int32),
            pltpu.VMEM((S,), jnp.float32),
            pltpu.VMEM((_NS * _NL,), jnp.float32),
            pltpu.VMEM((_NL,), jnp.float32),
            pltpu.VMEM_SHARED((_NS * _NL,), jnp.float32),
            pltpu.VMEM_SHARED((S,), jnp.float32),
        ],
    )(g_pad, ids2d)
    return m_out.reshape(_NC, S), s_out.reshape(_NC, S)


def _merge_cores(m_out, s_out):
    m_g = jnp.max(m_out)
    s_g = jnp.sum(s_out * jnp.exp(m_out - m_g), axis=0)
    return m_g, s_g


def kernel(nodes_atoms, nodes_bonds, nodes_monosacchs, batch_ids_atoms,
           batch_ids_bonds, batch_ids_monosacchs, gate_W, gate_b, W1, b1,
           prelu_a, bn_gamma, bn_beta, W2, b2):
    d = nodes_atoms.shape[1]
    na, nb, nm = (nodes_atoms.shape[0], nodes_bonds.shape[0],
                  nodes_monosacchs.shape[0])
    nba, nbb, nbm = na // R, nb // R, nm // R
    gw_col = gate_W.reshape(d, 1).astype(jnp.float32)
    gb11 = gate_b.reshape(1, 1).astype(jnp.float32)
    a11 = jnp.asarray(prelu_a, jnp.bfloat16).reshape(1, 1)
    b1_row = b1.reshape(1, d).astype(jnp.bfloat16)
    b2_row = b2.reshape(1, d)
    beta_row = bn_beta.reshape(1, d)
    gscale_row = (bn_gamma * (1.0 / jnp.sqrt(1.0 + 1e-5))).reshape(1, d)
    w1_bf = W1.astype(jnp.bfloat16)
    ids_a = batch_ids_atoms.astype(jnp.int32)
    ids_b = batch_ids_bonds.astype(jnp.int32)
    ids_m = batch_ids_monosacchs.astype(jnp.int32)

    blk = lambda shp, im: pl.BlockSpec(shp, im)
    row0 = lambda i: (i, 0)
    const0 = lambda i: (0, 0)
    rowm = lambda i: (jnp.minimum(i, nbm - 1), 0)

    gate_a, gate_m = pl.pallas_call(
        functools.partial(_k1_body, nbm),
        grid=(nba,),
        in_specs=[blk((R, d), row0), blk((R, d), rowm),
                  blk((d, 1), const0), blk((1, 1), const0)],
        out_specs=[blk((R, 1), row0), blk((R, 1), rowm)],
        out_shape=[jax.ShapeDtypeStruct((na, 1), jnp.float32),
                   jax.ShapeDtypeStruct((nm, 1), jnp.float32)],
        compiler_params=_ARB,
    )(nodes_atoms, nodes_monosacchs, gw_col, gb11)

    m_am_out, s_am_out = _sc_stats(
        jnp.concatenate([gate_a.reshape(-1), gate_m.reshape(-1)]),
        jnp.concatenate([ids_a, ids_m]))
    m_am, s_am = _merge_cores(m_am_out, s_am_out)
    m_am11 = m_am.reshape(1, 1)

    acc_a, gate_b_col = pl.pallas_call(
        _k2_body,
        grid=(nba,),
        in_specs=[blk((R, d), row0), blk((d, d), const0), blk((1, d), const0),
                  blk((1, 1), const0), blk((R, 1), row0), blk((R, 1), row0),
                  blk((1, 1), const0),
                  blk((R, d), row0), blk((d, 1), const0), blk((1, 1), const0)],
        out_specs=[blk((S, d), const0), blk((R, 1), row0)],
        out_shape=[jax.ShapeDtypeStruct((S, d), jnp.float32),
                   jax.ShapeDtypeStruct((nb, 1), jnp.float32)],
        compiler_params=_ARB,
    )(nodes_atoms, w1_bf, b1_row, a11, ids_a.reshape(na, 1), gate_a, m_am11,
      nodes_bonds, gw_col, gb11)

    m_b_out, s_b_out = _sc_stats(gate_b_col.reshape(-1), ids_b)
    m_b, s_b = _merge_cores(m_b_out, s_b_out)
    m_b11 = m_b.reshape(1, 1)

    acc_b, acc_m = pl.pallas_call(
        functools.partial(_k3_body, nbm),
        grid=(nbb,),
        in_specs=[blk((R, d), row0), blk((d, d), const0), blk((1, d), const0),
                  blk((1, 1), const0), blk((R, 1), row0), blk((R, 1), row0),
                  blk((1, 1), const0),
                  blk((R, d), rowm), blk((R, 1), rowm), blk((R, 1), rowm),
                  blk((1, 1), const0)],
        out_specs=[blk((S, d), const0), blk((S, d), const0)],
        out_shape=[jax.ShapeDtypeStruct((S, d), jnp.float32),
                   jax.ShapeDtypeStruct((S, d), jnp.float32)],
        compiler_params=_ARB,
    )(nodes_bonds, w1_bf, b1_row, a11, ids_b.reshape(nb, 1), gate_b_col,
      m_b11, nodes_monosacchs, ids_m.reshape(nm, 1), gate_m, m_am11)

    m_glob = jnp.maximum(m_am, m_b)
    k_am = jnp.exp(m_am - m_glob).reshape(1, 1)
    k_b = jnp.exp(m_b - m_glob).reshape(1, 1)
    den_col = (k_am[0, 0] * s_am + k_b[0, 0] * s_b).reshape(S, 1)

    out = pl.pallas_call(
        _epilogue_body,
        out_shape=jax.ShapeDtypeStruct((S, d), jnp.float32),
    )(acc_a, acc_b, acc_m, k_am, k_b, den_col, gscale_row, beta_row, W2,
      b2_row)
    return out
